# Initial kernel scaffold; baseline (speedup 1.0000x reference)
import jax
import jax.numpy as jnp
from jax.experimental import pallas as pl


def kernel(x, edge_index, W, att_src, att_dst, bias, t1_weight, t2_weight):
    raise NotImplementedError("write your pallas kernel here")


# speedup vs baseline: 2046.7139x; 2046.7139x over previous
import jax
import jax.numpy as jnp
from jax.experimental import pallas as pl

T, N, C = 8, 256, 256
_F32 = jnp.float32


def _body(x_ref, adj_ref, w_ref, asrc_ref, adst_ref, bias_ref, t1_ref, t2_ref,
          out_ref):
    w = w_ref[...]
    asrc = asrc_ref[...]
    adst = adst_ref[...]
    bias = bias_ref[...]
    t1 = t1_ref[...]
    t2 = t2_ref[...]

    row = jax.lax.broadcasted_iota(jnp.int32, (N, N), 0)
    col = jax.lax.broadcasted_iota(jnp.int32, (N, N), 1)
    diag = row == col

    outs = []
    for t in range(T):
        x_t = x_ref[t]
        h = jax.lax.dot_general(x_t, w, (((1,), (1,)), ((), ())),
                                preferred_element_type=_F32)
        a_src = jax.lax.dot_general(h, asrc, (((1,), (1,)), ((), ())),
                                    preferred_element_type=_F32)
        a_dst = jax.lax.dot_general(adst, h, (((1,), (1,)), ((), ())),
                                    preferred_element_type=_F32)
        z = a_src + a_dst
        e = jnp.where(z >= 0, z, 0.2 * z)
        valid = jnp.logical_or(diag, adj_ref[t] != 0)
        em = jnp.where(valid, e, -jnp.inf)
        emax = jnp.max(em, axis=0, keepdims=True)
        ee = jnp.exp(em - emax)
        den = jnp.sum(ee, axis=0, keepdims=True)
        coef = ee / den
        out_t = jax.lax.dot_general(coef, h, (((0,), (0,)), ((), ())),
                                    preferred_element_type=_F32)
        outs.append(out_t + bias)

    final = list(outs)
    for s in range(T - 1):
        a = outs[s]
        b = outs[s + 1]
        dot = jnp.sum(a * b, axis=1, keepdims=True)
        na = jnp.sqrt(jnp.sum(a * a, axis=1, keepdims=True))
        nb = jnp.sqrt(jnp.sum(b * b, axis=1, keepdims=True))
        tem = dot / jnp.maximum(na * nb, 1e-8)
        final[s + 1] = final[s + 1] + jax.lax.dot_general(
            tem * a, t2, (((1,), (0,)), ((), ())), preferred_element_type=_F32)
        final[s] = final[s] + jax.lax.dot_general(
            tem * b, t1, (((1,), (0,)), ((), ())), preferred_element_type=_F32)

    for t in range(T):
        out_ref[t] = final[t]


def kernel(x, edge_index, W, att_src, att_dst, bias, t1_weight, t2_weight):
    return pl.pallas_call(
        _body,
        out_shape=jax.ShapeDtypeStruct((T, N, C), jnp.float32),
    )(x, edge_index, W,
      att_src.reshape(1, C), att_dst.reshape(1, C), bias.reshape(1, C),
      t1_weight, t2_weight)
